# baseline (speedup 1.0000x reference)
import jax
import jax.numpy as jnp
from jax.experimental import pallas as pl
from jax.experimental.pallas import tpu as pltpu

QUEUE = 65536
FEAT = 128
BATCH = 4096
CH = 4096
NCH = QUEUE // CH


def _queue_body(ptr_smem, keys, data, out, ptr_out, bufs, lsem, ssem):
    praw = ptr_smem[0]
    pc = jnp.clip(praw, 0, QUEUE - BATCH)

    for i in range(NCH):
        g = i * CH
        in_keys = jnp.logical_and(g >= pc, g < pc + BATCH)

        @pl.when(in_keys)
        def _():
            pltpu.make_async_copy(
                keys.at[pl.ds(pl.multiple_of(g - pc, 8), CH)],
                bufs.at[pl.ds(g, CH)], lsem.at[i]).start()

        @pl.when(jnp.logical_not(in_keys))
        def _():
            pltpu.make_async_copy(
                data.at[pl.ds(g, CH)],
                bufs.at[pl.ds(g, CH)], lsem.at[i]).start()

    ptr_out[0] = (praw + BATCH) % QUEUE

    for i in range(NCH):
        g = i * CH
        pltpu.make_async_copy(
            data.at[pl.ds(0, CH)], bufs.at[pl.ds(g, CH)], lsem.at[i]).wait()
        pltpu.make_async_copy(
            bufs.at[pl.ds(g, CH)], out.at[pl.ds(g, CH)], ssem.at[i]).start()

    for i in range(NCH):
        g = i * CH
        pltpu.make_async_copy(
            bufs.at[pl.ds(g, CH)], out.at[pl.ds(g, CH)], ssem.at[i]).wait()


def kernel(keys, data, ptr):
    grid_spec = pltpu.PrefetchScalarGridSpec(
        num_scalar_prefetch=1,
        grid=(1,),
        in_specs=[
            pl.BlockSpec(memory_space=pl.ANY),
            pl.BlockSpec(memory_space=pl.ANY),
        ],
        out_specs=[
            pl.BlockSpec(memory_space=pl.ANY),
            pl.BlockSpec(memory_space=pltpu.SMEM),
        ],
        scratch_shapes=[
            pltpu.VMEM((QUEUE, FEAT), jnp.float32),
            pltpu.SemaphoreType.DMA((NCH,)),
            pltpu.SemaphoreType.DMA((NCH,)),
        ],
    )
    out, new_ptr = pl.pallas_call(
        _queue_body,
        grid_spec=grid_spec,
        out_shape=(
            jax.ShapeDtypeStruct((QUEUE, FEAT), jnp.float32),
            jax.ShapeDtypeStruct((1,), jnp.int32),
        ),
    )(ptr, keys, data)
    return out, new_ptr
